# Initial kernel scaffold; baseline (speedup 1.0000x reference)
import jax
import jax.numpy as jnp
from jax.experimental import pallas as pl


def kernel(x, edge_index, W1, b1, W2, b2, Wd, bd):
    raise NotImplementedError("write your pallas kernel here")


# speedup vs baseline: 13.3078x; 13.3078x over previous
import functools

import jax
import jax.numpy as jnp
from jax import lax
from jax.experimental import pallas as pl
from jax.experimental.pallas import tpu as pltpu
from jax.experimental.pallas import tpu_sc as plsc

N = 10000
E = 320000
D = 128
N_CLS = 16

NC = 2
NS = 16
NW = NC * NS
EPW = E // NW
CH = 80
NCH = EPW // CH
RPT = 624
REM_ROWS = N - NS * RPT

_mesh = plsc.VectorSubcoreMesh(core_axis_name="c", subcore_axis_name="s")


def _zero_vmem_2d(zb, nrows):
    def body(i, _):
        for j in range(D // 16):
            zb[i, pl.ds(j * 16, 16)] = jnp.zeros((16,), jnp.float32)
        return 0
    lax.fori_loop(0, nrows, body, 0)


@functools.partial(
    pl.kernel,
    out_type=jax.ShapeDtypeStruct((NC, N), jnp.float32),
    mesh=_mesh,
    scratch_types=[
        pltpu.VMEM((1, CH), jnp.int32),
        pltpu.VMEM((1, CH), jnp.float32),
        pltpu.VMEM((1024,), jnp.float32),
        pltpu.VMEM_SHARED((N,), jnp.float32),
    ],
)
def _deg_sc(dst_hbm, out_hbm, dstv, ones, zb, acc):
    cid = lax.axis_index("c")
    sid = lax.axis_index("s")
    w = sid * NC + cid

    def zb_body(i, _):
        zb[pl.ds(i * 16, 16)] = jnp.zeros((16,), jnp.float32)
        return 0
    lax.fori_loop(0, 64, zb_body, 0)
    for j in range(CH // 16):
        ones[0, pl.ds(j * 16, 16)] = jnp.full((16,), 1.0, jnp.float32)

    @pl.when(sid == 0)
    def _():
        def z_body(i, _):
            pltpu.sync_copy(zb, acc.at[pl.ds(i * 1024, 1024)])
            return 0
        lax.fori_loop(0, N // 1024, z_body, 0)
        pltpu.sync_copy(zb.at[pl.ds(0, N - (N // 1024) * 1024)],
                        acc.at[pl.ds((N // 1024) * 1024, N - (N // 1024) * 1024)])

    plsc.subcore_barrier()

    ebase = w * EPW

    def chunk(c, _):
        pltpu.sync_copy(dst_hbm.at[pl.ds(ebase + c * CH, CH)], dstv.at[0])
        pltpu.sync_copy(ones.at[0], acc.at[dstv.at[0]], add=True)
        return 0
    lax.fori_loop(0, NCH, chunk, 0)

    plsc.subcore_barrier()

    @pl.when(sid == 0)
    def _():
        pltpu.sync_copy(acc, out_hbm.at[cid])


@functools.partial(
    pl.kernel,
    out_type=jax.ShapeDtypeStruct((NC, N, D), jnp.float32),
    mesh=_mesh,
    scratch_types=[
        pltpu.VMEM((1, CH), jnp.int32),
        pltpu.VMEM((1, CH), jnp.int32),
        pltpu.VMEM((CH, D), jnp.float32),
        pltpu.VMEM((128, D), jnp.float32),
        pltpu.VMEM_SHARED((N, D), jnp.float32),
        pltpu.SemaphoreType.DMA,
    ],
)
def _msg_sc(y_hbm, src_hbm, dst_hbm, out_hbm, srcv, dstv, rows, zb, acc, gsem):
    cid = lax.axis_index("c")
    sid = lax.axis_index("s")
    w = sid * NC + cid

    _zero_vmem_2d(zb, 128)
    rbase = pl.multiple_of(sid * RPT, 8)
    rrem = RPT - (RPT // 128) * 128
    for j in range(RPT // 128):
        pltpu.sync_copy(zb, acc.at[pl.ds(rbase + j * 128, 128)])
    pltpu.sync_copy(zb.at[pl.ds(0, rrem)],
                    acc.at[pl.ds(rbase + (RPT // 128) * 128, rrem)])

    @pl.when(sid == 0)
    def _():
        pltpu.sync_copy(zb.at[pl.ds(0, REM_ROWS)],
                        acc.at[pl.ds(NS * RPT, REM_ROWS)])

    plsc.subcore_barrier()

    ebase = w * EPW

    def chunk(c, _):
        base = ebase + c * CH
        pltpu.sync_copy(src_hbm.at[pl.ds(base, CH)], srcv.at[0])
        pltpu.sync_copy(dst_hbm.at[pl.ds(base, CH)], dstv.at[0])
        pltpu.async_copy(y_hbm.at[srcv.at[0]], rows, gsem).wait()
        pltpu.sync_copy(rows, acc.at[dstv.at[0]], add=True)
        return 0
    lax.fori_loop(0, NCH, chunk, 0)

    plsc.subcore_barrier()

    for j in range(RPT // 128):
        pltpu.sync_copy(acc.at[pl.ds(rbase + j * 128, 128)],
                        out_hbm.at[cid, pl.ds(rbase + j * 128, 128)])
    pltpu.sync_copy(acc.at[pl.ds(rbase + (RPT // 128) * 128, rrem)],
                    out_hbm.at[cid, pl.ds(rbase + (RPT // 128) * 128, rrem)])

    @pl.when(sid == 0)
    def _():
        pltpu.sync_copy(acc.at[pl.ds(NS * RPT, REM_ROWS)],
                        out_hbm.at[cid, pl.ds(NS * RPT, REM_ROWS)])


_BR = 2000


def _dinv_block(degt_ref):
    deg = degt_ref[:, 0:1] + degt_ref[:, 1:2] + 1.0
    return lax.rsqrt(deg)


def _prescale1_body(degt_ref, x_ref, w_ref, y_ref):
    xw = jnp.dot(x_ref[...], w_ref[...], preferred_element_type=jnp.float32)
    y_ref[...] = xw * _dinv_block(degt_ref)


def _layer2_body(degt_ref, acc_ref, y_ref, b_ref, w_ref, out_ref):
    dinv = _dinv_block(degt_ref)
    s = acc_ref[0] + acc_ref[1] + y_ref[...]
    h = jnp.maximum(dinv * s + b_ref[...], 0.0)
    out_ref[...] = jnp.dot(h, w_ref[...], preferred_element_type=jnp.float32) * dinv


def _final_body(degt_ref, acc_ref, y_ref, b_ref, wd_ref, bd_ref, out_ref):
    dinv = _dinv_block(degt_ref)
    s = acc_ref[0] + acc_ref[1] + y_ref[...]
    h = jnp.maximum(dinv * s + b_ref[...], 0.0)
    logits = jnp.dot(h, wd_ref[...], preferred_element_type=jnp.float32) + bd_ref[...]
    m = jnp.max(logits, axis=1, keepdims=True)
    ex = jnp.exp(logits - m)
    out_ref[...] = ex / jnp.sum(ex, axis=1, keepdims=True)


def _row_spec(d):
    return pl.BlockSpec((_BR, d), lambda i: (i, 0))


def _full_spec(shape):
    return pl.BlockSpec(shape, lambda i: tuple(0 for _ in shape))


def kernel(x, edge_index, W1, b1, W2, b2, Wd, bd):
    ei = edge_index.astype(jnp.int32)
    src, dst = ei[0], ei[1]

    degp = _deg_sc(dst)
    degt = degp.T

    y1 = pl.pallas_call(
        _prescale1_body,
        grid=(N // _BR,),
        in_specs=[_row_spec(2), _row_spec(D), _full_spec((D, D))],
        out_specs=_row_spec(D),
        out_shape=jax.ShapeDtypeStruct((N, D), jnp.float32),
    )(degt, x, W1)

    acc1 = _msg_sc(y1, src, dst)

    y2 = pl.pallas_call(
        _layer2_body,
        grid=(N // _BR,),
        in_specs=[_row_spec(2),
                  pl.BlockSpec((2, _BR, D), lambda i: (0, i, 0)),
                  _row_spec(D), _full_spec((1, D)), _full_spec((D, D))],
        out_specs=_row_spec(D),
        out_shape=jax.ShapeDtypeStruct((N, D), jnp.float32),
    )(degt, acc1, y1, b1.reshape(1, D), W2)

    acc2 = _msg_sc(y2, src, dst)

    out = pl.pallas_call(
        _final_body,
        grid=(N // _BR,),
        in_specs=[_row_spec(2),
                  pl.BlockSpec((2, _BR, D), lambda i: (0, i, 0)),
                  _row_spec(D), _full_spec((1, D)), _full_spec((D, N_CLS)),
                  _full_spec((1, N_CLS))],
        out_specs=_row_spec(N_CLS),
        out_shape=jax.ShapeDtypeStruct((N, N_CLS), jnp.float32),
    )(degt, acc2, y2, b2.reshape(1, D), Wd, bd.reshape(1, N_CLS))

    return out
